# baseline (speedup 1.0000x reference)
import functools

import jax
import jax.numpy as jnp
from jax import lax
from jax.experimental import pallas as pl
from jax.experimental.pallas import tpu as pltpu
from jax.experimental.pallas import tpu_sc as plsc

_NC = 2
_NS = 16
_NW = _NC * _NS
_L = 16
_C = 128


@functools.lru_cache(maxsize=None)
def _build_tc_transpose(B, S, D):
    nq = S // 2

    def body(x_ref, o_ref):
        x = x_ref[...]
        o_ref[...] = x.reshape(2 * D, B).T

    return pl.pallas_call(
        body,
        grid=(nq,),
        in_specs=[pl.BlockSpec((2, D, B), lambda q: (q, 0, 0))],
        out_specs=pl.BlockSpec((B, 2 * D), lambda q: (q, 0)),
        out_shape=jax.ShapeDtypeStruct((nq * B, 2 * D), jnp.float32),
    )


@functools.lru_cache(maxsize=None)
def _build_sc_mse(N, D, n_chunks):
    n_pairs = n_chunks // 2
    mesh = plsc.VectorSubcoreMesh(core_axis_name="c", subcore_axis_name="s")

    @functools.partial(
        pl.kernel,
        mesh=mesh,
        compiler_params=pltpu.CompilerParams(use_tc_tiling_on_sc=False),
        out_type=jax.ShapeDtypeStruct((_NW, _L), jnp.float32),
        scratch_types=[
            pltpu.VMEM((n_chunks, _C), jnp.int32),
            pltpu.VMEM((_C, D), jnp.float32),
            pltpu.VMEM((_C, D), jnp.float32),
            pltpu.VMEM((_C, D), jnp.float32),
            pltpu.VMEM((_C, D), jnp.float32),
            pltpu.VMEM((_L,), jnp.float32),
            pltpu.SemaphoreType.DMA,
            pltpu.SemaphoreType.DMA,
            pltpu.SemaphoreType.DMA,
            pltpu.SemaphoreType.DMA,
        ],
    )
    def k(preds_hbm, idx_hbm, table_hbm, out_hbm,
          idx_all, p_a, p_b, r_a, r_b, acc_st, sp_a, sp_b, sr_a, sr_b):
        wid = lax.axis_index("s") * _NC + lax.axis_index("c")
        p_bufs = (p_a, p_b)
        r_bufs = (r_a, r_b)
        sp = (sp_a, sp_b)
        sr = (sr_a, sr_b)

        pltpu.sync_copy(idx_hbm.at[wid], idx_all)
        w_row0 = wid * (n_chunks * _C)

        def issue(c, b):
            rb = w_row0 + c * _C
            pltpu.async_copy(preds_hbm.at[pl.ds(rb, _C)], p_bufs[b], sp[b])
            pltpu.async_copy(table_hbm.at[idx_all.at[c]], r_bufs[b], sr[b])

        def wait(c, b):
            rb = w_row0 + c * _C
            pltpu.make_async_copy(
                preds_hbm.at[pl.ds(rb, _C)], p_bufs[b], sp[b]).wait()
            pltpu.make_async_copy(
                table_hbm.at[idx_all.at[c]], r_bufs[b], sr[b]).wait()

        def chunk_sum(b, accs):
            pv = p_bufs[b]
            rv = r_bufs[b]

            def row_body(r, accs):
                a = list(accs)
                for rr in range(2):
                    row = r * 2 + rr
                    for d in range(D // _L):
                        dp = (pv[row, pl.ds(d * _L, _L)]
                              - rv[row, pl.ds(d * _L, _L)])
                        a[d] = a[d] + dp * dp
                return tuple(a)

            return lax.fori_loop(0, _C // 2, row_body, accs)

        issue(0, 0)
        zero = jnp.zeros((_L,), jnp.float32)

        def pair_body(g, accs):
            issue(2 * g + 1, 1)
            wait(2 * g, 0)
            accs = chunk_sum(0, accs)

            @pl.when(g < n_pairs - 1)
            def _():
                issue(2 * g + 2, 0)

            wait(2 * g + 1, 1)
            accs = chunk_sum(1, accs)
            return accs

        accs = lax.fori_loop(0, n_pairs, pair_body, (zero, zero, zero, zero))
        acc_st[...] = (accs[0] + accs[1]) + (accs[2] + accs[3])
        pltpu.sync_copy(acc_st, out_hbm.at[wid])

    return k


def kernel(preds, target, table):
    B, S, D = preds.shape
    N = B * S
    per_w = N // _NW
    n_chunks = per_w // _C
    predsT = jnp.transpose(preds, (1, 2, 0))
    pp = _build_tc_transpose(B, S, D)(predsT)
    tgt2 = (target.T.reshape(S // 2, 2, B)
            .transpose(0, 2, 1)
            .reshape(_NW, n_chunks, _C))
    partials = _build_sc_mse(N, D, n_chunks)(
        pp.reshape(N, D),
        tgt2,
        table,
    )
    return jnp.sum(partials) * jnp.float32(1.0 / (N * D))


# speedup vs baseline: 1.2810x; 1.2810x over previous
import functools

import jax
import jax.numpy as jnp
from jax import lax
from jax.experimental import pallas as pl
from jax.experimental.pallas import tpu as pltpu
from jax.experimental.pallas import tpu_sc as plsc

_NC = 2
_NS = 16
_NW = _NC * _NS
_L = 16
_C = 128


@functools.lru_cache(maxsize=None)
def _build_tc_transpose(B, S, D):
    nq = S // 2

    def body(x_ref, o_ref):
        x = x_ref[...]
        o_ref[...] = x.reshape(2 * D, B).T

    return pl.pallas_call(
        body,
        grid=(nq,),
        in_specs=[pl.BlockSpec((2, D, B), lambda q: (q, 0, 0))],
        out_specs=pl.BlockSpec((B, 2 * D), lambda q: (q, 0)),
        out_shape=jax.ShapeDtypeStruct((nq * B, 2 * D), jnp.float32),
    )


@functools.lru_cache(maxsize=None)
def _build_sc_mse(B, S, D):
    nq = S // 2
    n_pairs = nq // 2
    mesh = plsc.VectorSubcoreMesh(core_axis_name="c", subcore_axis_name="s")

    @functools.partial(
        pl.kernel,
        mesh=mesh,
        compiler_params=pltpu.CompilerParams(use_tc_tiling_on_sc=False),
        out_type=jax.ShapeDtypeStruct((_NW, _L), jnp.float32),
        scratch_types=[
            pltpu.VMEM((S, _C), jnp.int32),
            pltpu.VMEM((_C, 2 * D), jnp.float32),
            pltpu.VMEM((_C, 2 * D), jnp.float32),
            pltpu.VMEM((_C, D), jnp.float32),
            pltpu.VMEM((_C, D), jnp.float32),
            pltpu.VMEM((_C, D), jnp.float32),
            pltpu.VMEM((_C, D), jnp.float32),
            pltpu.VMEM((_L,), jnp.float32),
            pltpu.SemaphoreType.DMA,
            pltpu.SemaphoreType.DMA,
            pltpu.SemaphoreType.DMA,
            pltpu.SemaphoreType.DMA,
            pltpu.SemaphoreType.DMA,
            pltpu.SemaphoreType.DMA,
        ],
    )
    def k(pp_hbm, tgtT_hbm, table_hbm, out_hbm,
          idx_all, p_a, p_b, ea_a, ea_b, eb_a, eb_b, acc_st,
          sp_a, sp_b, sa_a, sa_b, sb_a, sb_b):
        wid = lax.axis_index("s") * _NC + lax.axis_index("c")
        p_bufs = (p_a, p_b)
        ea_bufs = (ea_a, ea_b)
        eb_bufs = (eb_a, eb_b)
        sp = (sp_a, sp_b)
        sa = (sa_a, sa_b)
        sb = (sb_a, sb_b)

        b0 = wid * _C
        pltpu.sync_copy(tgtT_hbm.at[:, pl.ds(b0, _C)], idx_all)

        def issue(q, b):
            pltpu.async_copy(
                pp_hbm.at[pl.ds(q * B + b0, _C)], p_bufs[b], sp[b])
            pltpu.async_copy(
                table_hbm.at[idx_all.at[2 * q]], ea_bufs[b], sa[b])
            pltpu.async_copy(
                table_hbm.at[idx_all.at[2 * q + 1]], eb_bufs[b], sb[b])

        def wait(q, b):
            pltpu.make_async_copy(
                pp_hbm.at[pl.ds(q * B + b0, _C)], p_bufs[b], sp[b]).wait()
            pltpu.make_async_copy(
                table_hbm.at[idx_all.at[2 * q]], ea_bufs[b], sa[b]).wait()
            pltpu.make_async_copy(
                table_hbm.at[idx_all.at[2 * q + 1]], eb_bufs[b], sb[b]).wait()

        def chunk_sum(b, accs):
            pv = p_bufs[b]
            ea = ea_bufs[b]
            eb = eb_bufs[b]

            def row_body(r, accs):
                a = list(accs)
                for rr in range(2):
                    i = r * 2 + rr
                    for d in range(D // _L):
                        dpa = (pv[i, pl.ds(d * _L, _L)]
                               - ea[i, pl.ds(d * _L, _L)])
                        a[d] = a[d] + dpa * dpa
                        dpb = (pv[i, pl.ds(D + d * _L, _L)]
                               - eb[i, pl.ds(d * _L, _L)])
                        a[4 + d] = a[4 + d] + dpb * dpb
                return tuple(a)

            return lax.fori_loop(0, _C // 2, row_body, accs)

        issue(0, 0)
        zero = jnp.zeros((_L,), jnp.float32)

        def pair_body(g, accs):
            issue(2 * g + 1, 1)
            wait(2 * g, 0)
            accs = chunk_sum(0, accs)

            @pl.when(g < n_pairs - 1)
            def _():
                issue(2 * g + 2, 0)

            wait(2 * g + 1, 1)
            accs = chunk_sum(1, accs)
            return accs

        accs = lax.fori_loop(
            0, n_pairs, pair_body, (zero,) * 8)
        acc_st[...] = (((accs[0] + accs[1]) + (accs[2] + accs[3]))
                       + ((accs[4] + accs[5]) + (accs[6] + accs[7])))
        pltpu.sync_copy(acc_st, out_hbm.at[wid])

    return k


def kernel(preds, target, table):
    B, S, D = preds.shape
    N = B * S
    predsT = jnp.transpose(preds, (1, 2, 0))
    pp = _build_tc_transpose(B, S, D)(predsT)
    partials = _build_sc_mse(B, S, D)(
        pp,
        target.T,
        table,
    )
    return jnp.sum(partials) * jnp.float32(1.0 / (N * D))
